# Initial kernel scaffold; baseline (speedup 1.0000x reference)
import functools

import jax
import jax.numpy as jnp
from jax.experimental import pallas as pl
from jax.experimental.pallas import tpu as pltpu

N = 10000
F = 128
BN = 1000


def _dense_body(dis_ref, x_ref, h_ref, agg_ref,
                wz_ref, bz_ref, wr_ref, br_ref, wh_ref, bh_ref,
                wlz_ref, blz_ref, wlr_ref, blr_ref, wlh_ref, blh_ref,
                wout_ref, bout_ref, y_ref, hn_ref):
    d = dis_ref[:]
    x = x_ref[:]
    h = h_ref[:]
    p = d * (agg_ref[0] + agg_ref[1]) + (d * d) * x

    def mm(a, b):
        return jax.lax.dot_general(a, b, (((1,), (0,)), ((), ())),
                                   preferred_element_type=jnp.float32)

    cz = mm(p, wz_ref[:]) + bz_ref[:]
    cr = mm(p, wr_ref[:]) + br_ref[:]
    ch = mm(p, wh_ref[:]) + bh_ref[:]

    z = jax.nn.sigmoid(mm(cz, wlz_ref[:F]) + mm(h, wlz_ref[F:]) + blz_ref[:])
    r = jax.nn.sigmoid(mm(cr, wlr_ref[:F]) + mm(h, wlr_ref[F:]) + blr_ref[:])
    ht = jnp.tanh(mm(ch, wlh_ref[:F]) + mm(h * r, wlh_ref[F:]) + blh_ref[:])
    hn = z * h + (1.0 - z) * ht
    hn_ref[:] = hn
    y_ref[:] = mm(jnp.maximum(hn, 0.0), wout_ref[:]) + bout_ref[:]


def _dense_stage(dis, x, h, agg, Wz, bz, Wr, br, Wh, bh,
                 Wlz, blz, Wlr, blr, Wlh, blh, Wout, bout):
    grid = (N // BN,)
    row_spec = pl.BlockSpec((BN, F), lambda i: (i, 0))
    full = pl.BlockSpec((2, BN, F), lambda i: (0, i, 0))
    w_spec = pl.BlockSpec((F, F), lambda i: (0, 0))
    wl_spec = pl.BlockSpec((2 * F, F), lambda i: (0, 0))
    b_spec = pl.BlockSpec((1, F), lambda i: (0, 0))
    return pl.pallas_call(
        _dense_body,
        grid=grid,
        in_specs=[
            pl.BlockSpec((BN, 1), lambda i: (i, 0)),
            row_spec, row_spec, full,
            w_spec, b_spec, w_spec, b_spec, w_spec, b_spec,
            wl_spec, b_spec, wl_spec, b_spec, wl_spec, b_spec,
            w_spec, b_spec,
        ],
        out_specs=[row_spec, row_spec],
        out_shape=[jax.ShapeDtypeStruct((N, F), jnp.float32),
                   jax.ShapeDtypeStruct((N, F), jnp.float32)],
    )(dis, x, h, agg,
      Wz, bz.reshape(1, F), Wr, br.reshape(1, F), Wh, bh.reshape(1, F),
      Wlz, blz.reshape(1, F), Wlr, blr.reshape(1, F), Wlh, blh.reshape(1, F),
      Wout, bout.reshape(1, F))


def kernel(g, node_feat, edge_weight, hidden_state, Wz, bz, Wr, br, Wh, bh,
           Wlz, blz, Wlr, blr, Wlh, blh, Wout, bout):
    src, dst = g[0], g[1]
    x = node_feat

    deg = jnp.zeros((N,), x.dtype).at[dst].add(edge_weight) + 1.0
    dis = jax.lax.rsqrt(deg)
    scale = edge_weight * dis[src]
    agg = jnp.zeros((N, F), x.dtype).at[dst].add(x[src] * scale[:, None])
    agg2 = jnp.stack([agg, jnp.zeros_like(agg)])

    y, hn = _dense_stage(dis.reshape(N, 1), x, hidden_state, agg2,
                         Wz, bz, Wr, br, Wh, bh,
                         Wlz, blz, Wlr, blr, Wlh, blh, Wout, bout)
    return (y, hn)


# speedup vs baseline: 17.4380x; 17.4380x over previous
import functools

import jax
import jax.numpy as jnp
from jax import lax
from jax.experimental import pallas as pl
from jax.experimental.pallas import tpu as pltpu
from jax.experimental.pallas import tpu_sc as plsc

N = 10000
E = 320000
F = 128
BN = 1000

NC, NS = 2, 16
NPAD = 10240
NPT = NPAD // NS
EPT = E // (NC * NS)
EPT_DEG = E // NS
CB = 128
DEG_CH = EPT_DEG // CB
AGG_CH = EPT // CB


def _rsqrt_newton(d):
    i = lax.bitcast_convert_type(d, jnp.int32)
    i = jnp.int32(0x5F3759DF) - lax.shift_right_logical(i, 1)
    y = lax.bitcast_convert_type(i, jnp.float32)
    for _ in range(3):
        y = y * (1.5 - 0.5 * d * y * y)
    return y


def _sc_body(src_hbm, dst_hbm, ew_hbm, x_hbm, aggp_hbm, dis_hbm,
             srcv, dstv, ewv, rows, tilebuf, disfull, deg_idx,
             agg_sh, deg_sh, dis_sh, sem):
    cid = lax.axis_index("c")
    sid = lax.axis_index("s")
    wid = cid * NS + sid

    def zrow(r, _):
        for j in range(F // 16):
            tilebuf[r, pl.ds(j * 16, 16)] = jnp.zeros((16,), jnp.float32)
        return 0
    lax.fori_loop(0, CB, zrow, 0)
    for q in range(NPT // CB):
        pltpu.sync_copy(tilebuf, agg_sh.at[pl.ds(sid * NPT + q * CB, CB)])
    def zdeg(r, _):
        disfull[pl.ds(r * 16, 16)] = jnp.zeros((16,), jnp.float32)
        return 0
    lax.fori_loop(0, NPT // 16, zdeg, 0)
    pltpu.sync_copy(disfull.at[pl.ds(0, NPT)], deg_sh.at[pl.ds(sid * NPT, NPT)])
    plsc.subcore_barrier()

    def deg_chunk(base, n):
        pltpu.sync_copy(dst_hbm.at[pl.ds(base, n)], deg_idx.at[pl.ds(0, n)])
        pltpu.sync_copy(ew_hbm.at[pl.ds(base, n)], ewv.at[pl.ds(0, n)])
        pltpu.sync_copy(ewv.at[pl.ds(0, n)],
                        deg_sh.at[deg_idx.at[pl.ds(0, n)]], add=True)
    def deg_loop(k, _):
        deg_chunk(sid * EPT_DEG + k * CB, CB)
        return 0
    lax.fori_loop(0, DEG_CH, deg_loop, 0)
    rem = EPT_DEG - DEG_CH * CB
    if rem:
        deg_chunk(sid * EPT_DEG + DEG_CH * CB, rem)
    plsc.subcore_barrier()

    pltpu.sync_copy(deg_sh.at[pl.ds(sid * NPT, NPT)], disfull.at[pl.ds(0, NPT)])
    def dis_loop(r, _):
        d = disfull[pl.ds(r * 16, 16)] + 1.0
        disfull[pl.ds(r * 16, 16)] = _rsqrt_newton(d)
        return 0
    lax.fori_loop(0, NPT // 16, dis_loop, 0)
    pltpu.sync_copy(disfull.at[pl.ds(0, NPT)], dis_sh.at[pl.ds(sid * NPT, NPT)])
    @pl.when(cid == 0)
    def _():
        pltpu.sync_copy(disfull.at[pl.ds(0, NPT)], dis_hbm.at[pl.ds(sid * NPT, NPT)])
    plsc.subcore_barrier()
    pltpu.sync_copy(dis_sh, disfull)

    def agg_chunk(base, n):
        pltpu.sync_copy(src_hbm.at[pl.ds(base, n)], srcv.at[pl.ds(0, n)])
        pltpu.sync_copy(dst_hbm.at[pl.ds(base, n)], dstv.at[pl.ds(0, n)])
        pltpu.sync_copy(ew_hbm.at[pl.ds(base, n)], ewv.at[pl.ds(0, n)])
        pltpu.async_copy(x_hbm.at[srcv.at[pl.ds(0, n)]],
                         rows.at[pl.ds(0, n)], sem).wait()
        def scale_grp(k, _):
            s16 = (ewv[pl.ds(k * 16, 16)] *
                   plsc.load_gather(disfull, [srcv[pl.ds(k * 16, 16)]]))
            for i in range(16):
                s = s16[i]
                e = k * 16 + i
                for j in range(F // 16):
                    rows[e, pl.ds(j * 16, 16)] = rows[e, pl.ds(j * 16, 16)] * s
            return 0
        lax.fori_loop(0, n // 16, scale_grp, 0)
        pltpu.sync_copy(rows.at[pl.ds(0, n)],
                        agg_sh.at[dstv.at[pl.ds(0, n)]], add=True)
    def agg_loop(k, _):
        agg_chunk(wid * EPT + k * CB, CB)
        return 0
    lax.fori_loop(0, AGG_CH, agg_loop, 0)
    rem2 = EPT - AGG_CH * CB
    if rem2:
        agg_chunk(wid * EPT + AGG_CH * CB, rem2)
    plsc.subcore_barrier()

    for q in range(NPT // CB):
        off = sid * NPT + q * CB
        pltpu.sync_copy(agg_sh.at[pl.ds(off, CB)], rows)
        pltpu.sync_copy(rows, aggp_hbm.at[cid].at[pl.ds(off, CB)])


def _sparse_stage(src, dst, ew, x):
    mesh = plsc.VectorSubcoreMesh(core_axis_name="c", subcore_axis_name="s")
    f = pl.kernel(
        _sc_body,
        out_type=[jax.ShapeDtypeStruct((NC, NPAD, F), jnp.float32),
                  jax.ShapeDtypeStruct((NPAD,), jnp.float32)],
        mesh=mesh,
        scratch_types=[
            pltpu.VMEM((CB,), jnp.int32),
            pltpu.VMEM((CB,), jnp.int32),
            pltpu.VMEM((CB,), jnp.float32),
            pltpu.VMEM((CB, F), jnp.float32),
            pltpu.VMEM((CB, F), jnp.float32),
            pltpu.VMEM((NPAD,), jnp.float32),
            pltpu.VMEM((CB,), jnp.int32),
            pltpu.VMEM_SHARED((NPAD, F), jnp.float32),
            pltpu.VMEM_SHARED((NPAD,), jnp.float32),
            pltpu.VMEM_SHARED((NPAD,), jnp.float32),
            pltpu.SemaphoreType.DMA,
        ],
        compiler_params=pltpu.CompilerParams(needs_layout_passes=False),
    )
    return f(src, dst, ew, x)


def _dense_body(dis_ref, x_ref, h_ref, agg_ref,
                wz_ref, bz_ref, wr_ref, br_ref, wh_ref, bh_ref,
                wlz_ref, blz_ref, wlr_ref, blr_ref, wlh_ref, blh_ref,
                wout_ref, bout_ref, y_ref, hn_ref):
    d = dis_ref[:]
    x = x_ref[:]
    h = h_ref[:]
    p = d * (agg_ref[0] + agg_ref[1]) + (d * d) * x

    def mm(a, b):
        return jax.lax.dot_general(a, b, (((1,), (0,)), ((), ())),
                                   preferred_element_type=jnp.float32)

    cz = mm(p, wz_ref[:]) + bz_ref[:]
    cr = mm(p, wr_ref[:]) + br_ref[:]
    ch = mm(p, wh_ref[:]) + bh_ref[:]

    z = jax.nn.sigmoid(mm(cz, wlz_ref[:F]) + mm(h, wlz_ref[F:]) + blz_ref[:])
    r = jax.nn.sigmoid(mm(cr, wlr_ref[:F]) + mm(h, wlr_ref[F:]) + blr_ref[:])
    ht = jnp.tanh(mm(ch, wlh_ref[:F]) + mm(h * r, wlh_ref[F:]) + blh_ref[:])
    hn = z * h + (1.0 - z) * ht
    hn_ref[:] = hn
    y_ref[:] = mm(jnp.maximum(hn, 0.0), wout_ref[:]) + bout_ref[:]


def _dense_stage(dis, x, h, agg, Wz, bz, Wr, br, Wh, bh,
                 Wlz, blz, Wlr, blr, Wlh, blh, Wout, bout):
    grid = (N // BN,)
    row_spec = pl.BlockSpec((BN, F), lambda i: (i, 0))
    full = pl.BlockSpec((2, BN, F), lambda i: (0, i, 0))
    w_spec = pl.BlockSpec((F, F), lambda i: (0, 0))
    wl_spec = pl.BlockSpec((2 * F, F), lambda i: (0, 0))
    b_spec = pl.BlockSpec((1, F), lambda i: (0, 0))
    return pl.pallas_call(
        _dense_body,
        grid=grid,
        in_specs=[
            pl.BlockSpec((BN, 1), lambda i: (i, 0)),
            row_spec, row_spec, full,
            w_spec, b_spec, w_spec, b_spec, w_spec, b_spec,
            wl_spec, b_spec, wl_spec, b_spec, wl_spec, b_spec,
            w_spec, b_spec,
        ],
        out_specs=[row_spec, row_spec],
        out_shape=[jax.ShapeDtypeStruct((N, F), jnp.float32),
                   jax.ShapeDtypeStruct((N, F), jnp.float32)],
    )(dis, x, h, agg,
      Wz, bz.reshape(1, F), Wr, br.reshape(1, F), Wh, bh.reshape(1, F),
      Wlz, blz.reshape(1, F), Wlr, blr.reshape(1, F), Wlh, blh.reshape(1, F),
      Wout, bout.reshape(1, F))


def kernel(g, node_feat, edge_weight, hidden_state, Wz, bz, Wr, br, Wh, bh,
           Wlz, blz, Wlr, blr, Wlh, blh, Wout, bout):
    src, dst = g[0], g[1]
    x = node_feat

    aggp, dis_pad = _sparse_stage(src, dst, edge_weight, x)
    agg2 = aggp[:, :N, :]
    dis = dis_pad[:N]

    y, hn = _dense_stage(dis.reshape(N, 1), x, hidden_state, agg2,
                         Wz, bz, Wr, br, Wh, bh,
                         Wlz, blz, Wlr, blr, Wlh, blh, Wout, bout)
    return (y, hn)
